# baseline (speedup 1.0000x reference)
import functools

import jax
import jax.numpy as jnp
import numpy as np
from jax import lax
from jax.experimental import pallas as pl
from jax.experimental.pallas import tpu as pltpu
from jax.experimental.pallas import tpu_sc as plsc

_B, _T, _F, _H, _P, _NH = 2048, 32, 9, 64, 10000, 4
_DH = _H // _NH

_NC, _NS, _L = 2, 16, 16
_NW = _NC * _NS
_ROWS_PER_W = _B // _NW
_CHUNK = 4
_NCHUNK = _ROWS_PER_W // _CHUNK
_PB = (_P // 128) * 128


def _sc_gather_build():
  mesh = plsc.VectorSubcoreMesh(core_axis_name="c", subcore_axis_name="s")

  @functools.partial(
      pl.kernel,
      mesh=mesh,
      out_type=[
          jax.ShapeDtypeStruct((_B, _B), jnp.float32),
          jax.ShapeDtypeStruct((_B, _L), jnp.float32),
      ],
      scratch_types=[
          pltpu.VMEM((_B,), jnp.int32),
          pltpu.VMEM((_NCHUNK, _CHUNK), jnp.int32),
          pltpu.VMEM((_CHUNK, _PB), jnp.float32),
          pltpu.VMEM((_CHUNK, _PB), jnp.float32),
          pltpu.VMEM((_CHUNK, 128), jnp.float32),
          pltpu.VMEM((_CHUNK, 128), jnp.float32),
          pltpu.VMEM((_CHUNK, _B), jnp.float32),
          pltpu.VMEM((_CHUNK, _L), jnp.float32),
          pltpu.SemaphoreType.DMA,
      ],
      compiler_params=pltpu.CompilerParams(needs_layout_passes=False),
  )
  def sc_gather(sku_hbm, sku2d_hbm, adj_hbm, tail_hbm, g_hbm, degp_hbm,
                sku_v, myidx_v, rows_a, rows_b, tail_a, tail_b,
                g_v, degp_v, sem):
    wid = lax.axis_index("s") * _NC + lax.axis_index("c")
    base = wid * _ROWS_PER_W
    bufs = (rows_a, rows_b)
    tbufs = (tail_a, tail_b)

    pltpu.sync_copy(sku_hbm, sku_v)
    pltpu.sync_copy(sku2d_hbm.at[wid], myidx_v)

    def gather_copies(c, buf):
      idx = myidx_v.at[c]
      return (pltpu.make_async_copy(
                  adj_hbm.at[idx, pl.ds(0, _PB)], bufs[buf], sem),
              pltpu.make_async_copy(
                  tail_hbm.at[idx], tbufs[buf], sem))

    def start(c, buf):
      a, b = gather_copies(c, buf)
      a.start()
      b.start()

    def wait(c, buf):
      a, b = gather_copies(c, buf)
      a.wait()
      b.wait()

    start(0, 0)
    for c in range(_NCHUNK):
      buf = c % 2
      rows_v = bufs[buf]
      tail_v = tbufs[buf]
      wait(c, buf)
      if c + 1 < _NCHUNK:
        start(c + 1, 1 - buf)

      def do_row(i, _):
        def deg_body(j, acc):
          return acc + rows_v[i, pl.ds(j * _L, _L)]
        acc = lax.fori_loop(0, _PB // _L, deg_body,
                            jnp.zeros((_L,), jnp.float32), unroll=5)
        for j in range(128 // _L):
          acc = acc + tail_v[i, pl.ds(j * _L, _L)]
        degp_v[i, :] = acc

        row_sel = jnp.full((_L,), i, jnp.int32)

        def col_body(k, _):
          cols = sku_v[pl.ds(k * _L, _L)]
          in_body = cols < _PB
          bvals = plsc.load_gather(
              rows_v, [row_sel, jnp.minimum(cols, _PB - 1)])
          tvals = plsc.load_gather(
              tail_v, [row_sel, jnp.maximum(cols - _PB, 0)])
          g_v[i, pl.ds(k * _L, _L)] = jnp.where(in_body, bvals, tvals)
          return 0
        lax.fori_loop(0, _B // _L, col_body, 0, unroll=4)
        return 0

      lax.fori_loop(0, _CHUNK, do_row, 0)

      b0 = base + c * _CHUNK
      pltpu.sync_copy(g_v, g_hbm.at[pl.ds(b0, _CHUNK)])
      pltpu.sync_copy(degp_v, degp_hbm.at[pl.ds(b0, _CHUNK)])

  return sc_gather


def _sc_gather_call(sku, adj):
  sku2d = sku.reshape(_NW, _NCHUNK, _CHUNK)
  tail = jnp.pad(adj[:, _PB:], ((0, 0), (0, 128 - (_P - _PB))))
  return _sc_gather_build()(sku, sku2d, adj, tail)


def _lstm_body(x_ref, ws_ref, bs_ref, wvar_ref, w1i_ref, w1h_ref, b1_ref,
               w2i_ref, w2h_ref, b2_ref, out_ref,
               h1_ref, c1_ref, h2_ref, c2_ref):
  t = pl.program_id(0)

  @pl.when(t == 0)
  def _():
    z = jnp.zeros((_B, _H), jnp.float32)
    h1_ref[...] = z
    c1_ref[...] = z
    h2_ref[...] = z
    c2_ref[...] = z

  xt = x_ref[0]
  logits = jnp.dot(xt, ws_ref[...],
                   preferred_element_type=jnp.float32,
                   precision=lax.Precision.HIGHEST) + bs_ref[...]
  m = jnp.max(logits, axis=-1, keepdims=True)
  e = jnp.exp(logits - m)
  w = e / jnp.sum(e, axis=-1, keepdims=True)
  vt = jnp.zeros((_B, _H), jnp.float32)
  for f in range(_F):
    vt = vt + w[:, f:f + 1] * (xt[:, f:f + 1] * wvar_ref[f:f + 1, :])

  def cell_update(g, c):
    i = jax.nn.sigmoid(g[:, 0 * _H:1 * _H])
    f = jax.nn.sigmoid(g[:, 1 * _H:2 * _H])
    gg = jnp.tanh(g[:, 2 * _H:3 * _H])
    o = jax.nn.sigmoid(g[:, 3 * _H:4 * _H])
    cn = f * c + i * gg
    hn = o * jnp.tanh(cn)
    return hn, cn

  hp = lambda a, b: jnp.dot(a, b, preferred_element_type=jnp.float32,
                            precision=lax.Precision.HIGHEST)
  g1 = hp(vt, w1i_ref[...]) + hp(h1_ref[...], w1h_ref[...]) + b1_ref[...]
  h1, c1 = cell_update(g1, c1_ref[...])
  h1_ref[...] = h1
  c1_ref[...] = c1

  g2 = hp(h1, w2i_ref[...]) + hp(h2_ref[...], w2h_ref[...]) + b2_ref[...]
  h2, c2 = cell_update(g2, c2_ref[...])
  h2_ref[...] = h2
  c2_ref[...] = c2
  out_ref[0] = h2


def _lstm_call(xw, ws, bs, wvar, w1i, w1h, b1, w2i, w2h, b2):
  full = lambda shape: pl.BlockSpec(shape, lambda t: (0,) * len(shape))
  return pl.pallas_call(
      _lstm_body,
      grid=(_T,),
      in_specs=[
          pl.BlockSpec((1, _B, _F), lambda t: (t, 0, 0)),
          full((_F, _F)), full((1, _F)), full((_F, _H)),
          full((_H, 4 * _H)), full((_H, 4 * _H)), full((1, 4 * _H)),
          full((_H, 4 * _H)), full((_H, 4 * _H)), full((1, 4 * _H)),
      ],
      out_specs=pl.BlockSpec((1, _B, _H), lambda t: (t, 0, 0)),
      out_shape=jax.ShapeDtypeStruct((_T, _B, _H), jnp.float32),
      scratch_shapes=[pltpu.VMEM((_B, _H), jnp.float32)] * 4,
      compiler_params=pltpu.CompilerParams(
          dimension_semantics=("arbitrary",)),
  )(xw, ws, bs, wvar, w1i, w1h, b1, w2i, w2h, b2)


_MHA_BLK = 256


def _mha_body(h2_ref, wq_ref, bq_ref, wk_ref, bk_ref, wv_ref, bv_ref,
              wo_ref, bo_ref, wout_ref, bout_ref, hid_ref, tft_ref):
  h2 = h2_ref[...]
  h2f = h2.reshape(_T * _MHA_BLK, _H)
  k = jnp.dot(h2f, wk_ref[...],
              preferred_element_type=jnp.float32, precision=lax.Precision.HIGHEST) + bk_ref[...]
  v = jnp.dot(h2f, wv_ref[...],
              preferred_element_type=jnp.float32, precision=lax.Precision.HIGHEST) + bv_ref[...]
  q = jnp.dot(h2[_T - 1], wq_ref[...],
              preferred_element_type=jnp.float32, precision=lax.Precision.HIGHEST) + bq_ref[...]

  k3 = k.reshape(_T, _MHA_BLK, _H)
  v3 = v.reshape(_T, _MHA_BLK, _H)
  scale = 1.0 / np.sqrt(_DH)

  outs = []
  for n in range(_NH):
    sl = slice(n * _DH, (n + 1) * _DH)
    kn = k3[:, :, sl]
    qn = q[:, sl]
    s = jnp.sum(kn * qn[None], axis=-1) * scale
    m = jnp.max(s, axis=0, keepdims=True)
    e = jnp.exp(s - m)
    p = e / jnp.sum(e, axis=0, keepdims=True)
    on = jnp.sum(v3[:, :, sl] * p[:, :, None], axis=0)
    outs.append(on)
  o = jnp.concatenate(outs, axis=-1)
  hid = jnp.dot(o, wo_ref[...],
                preferred_element_type=jnp.float32, precision=lax.Precision.HIGHEST) + bo_ref[...]
  hid_ref[...] = hid
  tft_ref[...] = (jnp.sum(hid * wout_ref[...], axis=-1, keepdims=True)
                  + bout_ref[...])


def _mha_call(h2seq, wq, bq, wk, bk, wv, bv, wo, bo, wout_row, bout):
  full = lambda shape: pl.BlockSpec(shape, lambda c: (0,) * len(shape))
  nblk = _B // _MHA_BLK
  return pl.pallas_call(
      _mha_body,
      grid=(nblk,),
      in_specs=[
          pl.BlockSpec((_T, _MHA_BLK, _H), lambda c: (0, c, 0)),
          full((_H, _H)), full((1, _H)),
          full((_H, _H)), full((1, _H)),
          full((_H, _H)), full((1, _H)),
          full((_H, _H)), full((1, _H)),
          full((1, _H)), full((1, 1)),
      ],
      out_specs=[
          pl.BlockSpec((_MHA_BLK, _H), lambda c: (c, 0)),
          pl.BlockSpec((_MHA_BLK, 1), lambda c: (c, 0)),
      ],
      out_shape=[
          jax.ShapeDtypeStruct((_B, _H), jnp.float32),
          jax.ShapeDtypeStruct((_B, 1), jnp.float32),
      ],
      compiler_params=pltpu.CompilerParams(
          dimension_semantics=("arbitrary",)),
  )(h2seq, wq, bq, wk, bk, wv, bv, wo, bo, wout_row, bout)


_FIN_BLK = 128
_WM_CHUNK = 256


def _final_body(g_ref, degp_ref, hid_ref, win_ref, tft_ref,
                wg_ref, bg_ref, lng_ref, lnb_ref, wgo_ref, bgo_ref,
                alpha_ref, out_ref, hw_ref):
  pid = pl.program_id(0)

  @pl.when(pid == 0)
  def _():
    bid = lax.broadcasted_iota(jnp.int32, (_B, 1), 0)
    wm = jnp.where(win_ref[...] == bid, 1.0, 0.0)
    hw_ref[...] = hid_ref[...] * wm

  deg = jnp.clip(jnp.sum(degp_ref[...], axis=-1, keepdims=True), 1e-6, None)
  gn = g_ref[...] / deg
  agg = jnp.dot(gn, hw_ref[...],
                preferred_element_type=jnp.float32,
                precision=lax.Precision.HIGHEST)
  tt = jnp.dot(agg, wg_ref[...],
               preferred_element_type=jnp.float32, precision=lax.Precision.HIGHEST) + bg_ref[...]
  tt = 0.5 * tt * (1.0 + lax.erf(tt * np.float32(1.0 / np.sqrt(2.0))))
  mu = jnp.mean(tt, axis=-1, keepdims=True)
  d = tt - mu
  var = jnp.mean(d * d, axis=-1, keepdims=True)
  tn = d / jnp.sqrt(var + 1e-5) * lng_ref[...] + lnb_ref[...]
  gnn = jnp.sum(tn * wgo_ref[...], axis=-1, keepdims=True) + bgo_ref[...]
  a = alpha_ref[...]
  out_ref[...] = a * tft_ref[...] + (1.0 - a) * gnn


def _final_call(g, degp, hidden, winner, tft, wg, bg, lng, lnb,
                wgo_row, bgo, alpha_sig):
  full = lambda shape: pl.BlockSpec(shape, lambda i: (0,) * len(shape))
  nblk = _B // _FIN_BLK
  return pl.pallas_call(
      _final_body,
      grid=(nblk,),
      in_specs=[
          pl.BlockSpec((_FIN_BLK, _B), lambda i: (i, 0)),
          pl.BlockSpec((_FIN_BLK, _L), lambda i: (i, 0)),
          full((_B, _H)), full((_B, 1)),
          pl.BlockSpec((_FIN_BLK, 1), lambda i: (i, 0)),
          full((_H, _H)), full((1, _H)), full((1, _H)), full((1, _H)),
          full((1, _H)), full((1, 1)), full((1, 1)),
      ],
      out_specs=pl.BlockSpec((_FIN_BLK, 1), lambda i: (i, 0)),
      out_shape=jax.ShapeDtypeStruct((_B, 1), jnp.float32),
      scratch_shapes=[pltpu.VMEM((_B, _H), jnp.float32)],
      compiler_params=pltpu.CompilerParams(
          dimension_semantics=("arbitrary",)),
  )(g, degp, hidden, winner, tft, wg, bg, lng, lnb, wgo_row,
    bgo, alpha_sig)


def kernel(x, sku_indices, adj, params):
  p = params
  xw = jnp.transpose(x, (1, 0, 2))
  sku = sku_indices.astype(jnp.int32)

  ws = p['Ws']
  bs = p['bs'].reshape(1, _F)
  b1 = (p['bih0'] + p['bhh0']).reshape(1, 4 * _H)
  b2 = (p['bih1'] + p['bhh1']).reshape(1, 4 * _H)

  g_mat, degp = _sc_gather_call(sku, adj)

  h2seq = _lstm_call(xw, ws, bs, p['W_var'],
                     p['Wih0'].T, p['Whh0'].T, b1,
                     p['Wih1'].T, p['Whh1'].T, b2)

  hidden, tft = _mha_call(
      h2seq, p['Wq'], p['bq'].reshape(1, _H), p['Wk'], p['bk'].reshape(1, _H),
      p['Wv'], p['bv'].reshape(1, _H), p['Wo'], p['bo'].reshape(1, _H),
      p['Wout'].reshape(1, _H), p['bout'].reshape(1, 1))

  valid = sku >= 0
  safe_idx = jnp.where(valid, sku, _P)
  pe = jnp.zeros((_P, _H), jnp.float32).at[safe_idx].set(hidden, mode='drop')
  wmask = jnp.all(pe[sku] == hidden, axis=1)
  winner = jnp.where(wmask, jnp.arange(_B, dtype=jnp.int32),
                     -1).reshape(_B, 1)

  alpha_sig = jax.nn.sigmoid(p['alpha']).reshape(1, 1)
  out2d = _final_call(
      g_mat, degp, hidden, winner, tft,
      p['Wg'], p['bg'].reshape(1, _H), p['ln_g'].reshape(1, _H),
      p['ln_b'].reshape(1, _H), p['Wgo'].reshape(1, _H),
      p['bgo'].reshape(1, 1), alpha_sig)
  return out2d[:, 0]


# speedup vs baseline: 1.2034x; 1.2034x over previous
import functools

import jax
import jax.numpy as jnp
import numpy as np
from jax import lax
from jax.experimental import pallas as pl
from jax.experimental.pallas import tpu as pltpu
from jax.experimental.pallas import tpu_sc as plsc

_B, _T, _F, _H, _P, _NH = 2048, 32, 9, 64, 10000, 4
_DH = _H // _NH

_NC, _NS, _L = 2, 16, 16
_NW = _NC * _NS
_ROWS_PER_W = _B // _NW
_CHUNK = 4
_NCHUNK = _ROWS_PER_W // _CHUNK
_PB = (_P // 128) * 128


def _sc_gather_build():
  mesh = plsc.VectorSubcoreMesh(core_axis_name="c", subcore_axis_name="s")

  @functools.partial(
      pl.kernel,
      mesh=mesh,
      out_type=[
          jax.ShapeDtypeStruct((_B, _B), jnp.float32),
          jax.ShapeDtypeStruct((_B, _L), jnp.float32),
      ],
      scratch_types=[
          pltpu.VMEM((_B,), jnp.int32),
          pltpu.VMEM((_NCHUNK, _CHUNK), jnp.int32),
          pltpu.VMEM((_CHUNK, _PB), jnp.float32),
          pltpu.VMEM((_CHUNK, _PB), jnp.float32),
          pltpu.VMEM((_CHUNK, 128), jnp.float32),
          pltpu.VMEM((_CHUNK, 128), jnp.float32),
          pltpu.VMEM((_CHUNK, _B), jnp.float32),
          pltpu.VMEM((_CHUNK, _L), jnp.float32),
          pltpu.SemaphoreType.DMA,
      ],
      compiler_params=pltpu.CompilerParams(needs_layout_passes=False),
  )
  def sc_gather(sku_hbm, sku2d_hbm, adj_hbm, tail_hbm, g_hbm, degp_hbm,
                sku_v, myidx_v, rows_a, rows_b, tail_a, tail_b,
                g_v, degp_v, sem):
    wid = lax.axis_index("s") * _NC + lax.axis_index("c")
    base = wid * _ROWS_PER_W
    bufs = (rows_a, rows_b)
    tbufs = (tail_a, tail_b)

    pltpu.sync_copy(sku_hbm, sku_v)
    pltpu.sync_copy(sku2d_hbm.at[wid], myidx_v)

    def gather_copies(c, buf):
      idx = myidx_v.at[c]
      return (pltpu.make_async_copy(
                  adj_hbm.at[idx, pl.ds(0, _PB)], bufs[buf], sem),
              pltpu.make_async_copy(
                  tail_hbm.at[idx], tbufs[buf], sem))

    def start(c, buf):
      a, b = gather_copies(c, buf)
      a.start()
      b.start()

    def wait(c, buf):
      a, b = gather_copies(c, buf)
      a.wait()
      b.wait()

    start(0, 0)
    for c in range(_NCHUNK):
      buf = c % 2
      rows_v = bufs[buf]
      tail_v = tbufs[buf]
      wait(c, buf)
      if c + 1 < _NCHUNK:
        start(c + 1, 1 - buf)

      def do_row(i, _):
        def deg_body(j, acc):
          return acc + rows_v[i, pl.ds(j * _L, _L)]
        acc = lax.fori_loop(0, _PB // _L, deg_body,
                            jnp.zeros((_L,), jnp.float32), unroll=5)
        for j in range(128 // _L):
          acc = acc + tail_v[i, pl.ds(j * _L, _L)]
        degp_v[i, :] = acc

        row_sel = jnp.full((_L,), i, jnp.int32)

        def col_body(k, _):
          cols = sku_v[pl.ds(k * _L, _L)]
          in_body = cols < _PB
          bvals = plsc.load_gather(
              rows_v, [row_sel, jnp.minimum(cols, _PB - 1)])
          tvals = plsc.load_gather(
              tail_v, [row_sel, jnp.maximum(cols - _PB, 0)])
          g_v[i, pl.ds(k * _L, _L)] = jnp.where(in_body, bvals, tvals)
          return 0
        lax.fori_loop(0, _B // _L, col_body, 0, unroll=4)
        return 0

      lax.fori_loop(0, _CHUNK, do_row, 0)

      b0 = base + c * _CHUNK
      pltpu.sync_copy(g_v, g_hbm.at[pl.ds(b0, _CHUNK)])
      pltpu.sync_copy(degp_v, degp_hbm.at[pl.ds(b0, _CHUNK)])

  return sc_gather


def _sc_gather_call(sku, adj):
  sku2d = sku.reshape(_NW, _NCHUNK, _CHUNK)
  tail = jnp.pad(adj[:, _PB:], ((0, 0), (0, 128 - (_P - _PB))))
  return _sc_gather_build()(sku, sku2d, adj, tail)


def _lstm_body(x_ref, ws_ref, bs_ref, wvar_ref, w1i_ref, w1h_ref, b1_ref,
               w2i_ref, w2h_ref, b2_ref, out_ref,
               h1_ref, c1_ref, h2_ref, c2_ref):
  t = pl.program_id(0)

  @pl.when(t == 0)
  def _():
    z = jnp.zeros((_B, _H), jnp.float32)
    h1_ref[...] = z
    c1_ref[...] = z
    h2_ref[...] = z
    c2_ref[...] = z

  xt = x_ref[0]
  logits = jnp.dot(xt, ws_ref[...],
                   preferred_element_type=jnp.float32,
                   precision=lax.Precision.HIGHEST) + bs_ref[...]
  m = jnp.max(logits, axis=-1, keepdims=True)
  e = jnp.exp(logits - m)
  w = e / jnp.sum(e, axis=-1, keepdims=True)
  vt = jnp.zeros((_B, _H), jnp.float32)
  for f in range(_F):
    vt = vt + w[:, f:f + 1] * (xt[:, f:f + 1] * wvar_ref[f:f + 1, :])

  def cell_update(g, c):
    i = jax.nn.sigmoid(g[:, 0 * _H:1 * _H])
    f = jax.nn.sigmoid(g[:, 1 * _H:2 * _H])
    gg = jnp.tanh(g[:, 2 * _H:3 * _H])
    o = jax.nn.sigmoid(g[:, 3 * _H:4 * _H])
    cn = f * c + i * gg
    hn = o * jnp.tanh(cn)
    return hn, cn

  hp = lambda a, b: jnp.dot(a, b, preferred_element_type=jnp.float32,
                            precision=lax.Precision.HIGHEST)
  g1 = hp(vt, w1i_ref[...]) + hp(h1_ref[...], w1h_ref[...]) + b1_ref[...]
  h1, c1 = cell_update(g1, c1_ref[...])
  h1_ref[...] = h1
  c1_ref[...] = c1

  g2 = hp(h1, w2i_ref[...]) + hp(h2_ref[...], w2h_ref[...]) + b2_ref[...]
  h2, c2 = cell_update(g2, c2_ref[...])
  h2_ref[...] = h2
  c2_ref[...] = c2
  out_ref[0] = h2


def _lstm_call(xw, ws, bs, wvar, w1i, w1h, b1, w2i, w2h, b2):
  full = lambda shape: pl.BlockSpec(shape, lambda t: (0,) * len(shape))
  return pl.pallas_call(
      _lstm_body,
      grid=(_T,),
      in_specs=[
          pl.BlockSpec((1, _B, _F), lambda t: (t, 0, 0)),
          full((_F, _F)), full((1, _F)), full((_F, _H)),
          full((_H, 4 * _H)), full((_H, 4 * _H)), full((1, 4 * _H)),
          full((_H, 4 * _H)), full((_H, 4 * _H)), full((1, 4 * _H)),
      ],
      out_specs=pl.BlockSpec((1, _B, _H), lambda t: (t, 0, 0)),
      out_shape=jax.ShapeDtypeStruct((_T, _B, _H), jnp.float32),
      scratch_shapes=[pltpu.VMEM((_B, _H), jnp.float32)] * 4,
      compiler_params=pltpu.CompilerParams(
          dimension_semantics=("arbitrary",)),
  )(xw, ws, bs, wvar, w1i, w1h, b1, w2i, w2h, b2)


_MHA_BLK = 256


def _mha_body(h2_ref, wq_ref, bq_ref, wk_ref, bk_ref, wv_ref, bv_ref,
              wo_ref, bo_ref, wout_ref, bout_ref, hid_ref, tft_ref):
  h2 = h2_ref[...]
  h2f = h2.reshape(_T * _MHA_BLK, _H)
  k = jnp.dot(h2f, wk_ref[...],
              preferred_element_type=jnp.float32, precision=lax.Precision.HIGHEST) + bk_ref[...]
  v = jnp.dot(h2f, wv_ref[...],
              preferred_element_type=jnp.float32, precision=lax.Precision.HIGHEST) + bv_ref[...]
  q = jnp.dot(h2[_T - 1], wq_ref[...],
              preferred_element_type=jnp.float32, precision=lax.Precision.HIGHEST) + bq_ref[...]

  k3 = k.reshape(_T, _MHA_BLK, _H)
  v3 = v.reshape(_T, _MHA_BLK, _H)
  scale = 1.0 / np.sqrt(_DH)

  outs = []
  for n in range(_NH):
    sl = slice(n * _DH, (n + 1) * _DH)
    kn = k3[:, :, sl]
    qn = q[:, sl]
    s = jnp.sum(kn * qn[None], axis=-1) * scale
    m = jnp.max(s, axis=0, keepdims=True)
    e = jnp.exp(s - m)
    p = e / jnp.sum(e, axis=0, keepdims=True)
    on = jnp.sum(v3[:, :, sl] * p[:, :, None], axis=0)
    outs.append(on)
  o = jnp.concatenate(outs, axis=-1)
  hid = jnp.dot(o, wo_ref[...],
                preferred_element_type=jnp.float32, precision=lax.Precision.HIGHEST) + bo_ref[...]
  hid_ref[...] = hid
  tft_ref[...] = (jnp.sum(hid * wout_ref[...], axis=-1, keepdims=True)
                  + bout_ref[...])


def _mha_call(h2seq, wq, bq, wk, bk, wv, bv, wo, bo, wout_row, bout):
  full = lambda shape: pl.BlockSpec(shape, lambda c: (0,) * len(shape))
  nblk = _B // _MHA_BLK
  return pl.pallas_call(
      _mha_body,
      grid=(nblk,),
      in_specs=[
          pl.BlockSpec((_T, _MHA_BLK, _H), lambda c: (0, c, 0)),
          full((_H, _H)), full((1, _H)),
          full((_H, _H)), full((1, _H)),
          full((_H, _H)), full((1, _H)),
          full((_H, _H)), full((1, _H)),
          full((1, _H)), full((1, 1)),
      ],
      out_specs=[
          pl.BlockSpec((_MHA_BLK, _H), lambda c: (c, 0)),
          pl.BlockSpec((_MHA_BLK, 1), lambda c: (c, 0)),
      ],
      out_shape=[
          jax.ShapeDtypeStruct((_B, _H), jnp.float32),
          jax.ShapeDtypeStruct((_B, 1), jnp.float32),
      ],
      compiler_params=pltpu.CompilerParams(
          dimension_semantics=("arbitrary",)),
  )(h2seq, wq, bq, wk, bk, wv, bv, wo, bo, wout_row, bout)


_FIN_BLK = 128
_WM_CHUNK = 256


def _final_body(g_ref, degp_ref, hid_ref, win_ref, tft_ref,
                wg_ref, bg_ref, lng_ref, lnb_ref, wgo_ref, bgo_ref,
                alpha_ref, out_ref, hw_ref):
  pid = pl.program_id(0)

  @pl.when(pid == 0)
  def _():
    bid = lax.broadcasted_iota(jnp.int32, (_B, 1), 0)
    wm = jnp.where(win_ref[...] == bid, 1.0, 0.0)
    hw_ref[...] = hid_ref[...] * wm

  deg = jnp.clip(jnp.sum(degp_ref[...], axis=-1, keepdims=True), 1e-6, None)
  gn = g_ref[...] / deg
  agg = jnp.dot(gn, hw_ref[...],
                preferred_element_type=jnp.float32,
                precision=lax.Precision.HIGHEST)
  tt = jnp.dot(agg, wg_ref[...],
               preferred_element_type=jnp.float32, precision=lax.Precision.HIGHEST) + bg_ref[...]
  tt = 0.5 * tt * (1.0 + lax.erf(tt * np.float32(1.0 / np.sqrt(2.0))))
  mu = jnp.mean(tt, axis=-1, keepdims=True)
  d = tt - mu
  var = jnp.mean(d * d, axis=-1, keepdims=True)
  tn = d / jnp.sqrt(var + 1e-5) * lng_ref[...] + lnb_ref[...]
  gnn = jnp.sum(tn * wgo_ref[...], axis=-1, keepdims=True) + bgo_ref[...]
  a = alpha_ref[...]
  out_ref[...] = a * tft_ref[...] + (1.0 - a) * gnn


def _final_call(g, degp, hidden, winner, tft, wg, bg, lng, lnb,
                wgo_row, bgo, alpha_sig):
  full = lambda shape: pl.BlockSpec(shape, lambda i: (0,) * len(shape))
  nblk = _B // _FIN_BLK
  return pl.pallas_call(
      _final_body,
      grid=(nblk,),
      in_specs=[
          pl.BlockSpec((_FIN_BLK, _B), lambda i: (i, 0)),
          pl.BlockSpec((_FIN_BLK, _L), lambda i: (i, 0)),
          full((_B, _H)), full((_B, 1)),
          pl.BlockSpec((_FIN_BLK, 1), lambda i: (i, 0)),
          full((_H, _H)), full((1, _H)), full((1, _H)), full((1, _H)),
          full((1, _H)), full((1, 1)), full((1, 1)),
      ],
      out_specs=pl.BlockSpec((_FIN_BLK, 1), lambda i: (i, 0)),
      out_shape=jax.ShapeDtypeStruct((_B, 1), jnp.float32),
      scratch_shapes=[pltpu.VMEM((_B, _H), jnp.float32)],
      compiler_params=pltpu.CompilerParams(
          dimension_semantics=("arbitrary",)),
  )(g, degp, hidden, winner, tft, wg, bg, lng, lnb, wgo_row,
    bgo, alpha_sig)


def kernel(x, sku_indices, adj, params):
  p = params
  xw = jnp.transpose(x, (1, 0, 2))
  sku = sku_indices.astype(jnp.int32)

  ws = p['Ws']
  bs = p['bs'].reshape(1, _F)
  b1 = (p['bih0'] + p['bhh0']).reshape(1, 4 * _H)
  b2 = (p['bih1'] + p['bhh1']).reshape(1, 4 * _H)

  g_mat, degp = _sc_gather_call(sku, adj)

  h2seq = _lstm_call(xw, ws, bs, p['W_var'],
                     p['Wih0'].T, p['Whh0'].T, b1,
                     p['Wih1'].T, p['Whh1'].T, b2)

  hidden, tft = _mha_call(
      h2seq, p['Wq'], p['bq'].reshape(1, _H), p['Wk'], p['bk'].reshape(1, _H),
      p['Wv'], p['bv'].reshape(1, _H), p['Wo'], p['bo'].reshape(1, _H),
      p['Wout'].reshape(1, _H), p['bout'].reshape(1, 1))

  valid = sku >= 0
  safe_idx = jnp.where(valid, sku, _P)
  barr = lax.broadcasted_iota(jnp.float32, (_B, _H), 0)
  pe = jnp.zeros((_P, _H), jnp.float32).at[safe_idx].set(barr, mode='drop')
  winner = pe[sku, 0].astype(jnp.int32).reshape(_B, 1)

  alpha_sig = jax.nn.sigmoid(p['alpha']).reshape(1, 1)
  out2d = _final_call(
      g_mat, degp, hidden, winner, tft,
      p['Wg'], p['bg'].reshape(1, _H), p['ln_g'].reshape(1, _H),
      p['ln_b'].reshape(1, _H), p['Wgo'].reshape(1, _H),
      p['bgo'].reshape(1, 1), alpha_sig)
  return out2d[:, 0]
